# baseline (speedup 1.0000x reference)
import functools

import jax
import jax.numpy as jnp
from jax import lax
from jax.experimental import pallas as pl
from jax.experimental.pallas import tpu as pltpu
from jax.experimental.pallas import tpu_sc as plsc

B = 16384
D = 32
H1 = 128
H2 = 64

_NC = 2
_NS = 16
_NW = _NC * _NS
_BPW = B // _NW

_mesh = plsc.VectorSubcoreMesh(core_axis_name="c", subcore_axis_name="s")


@functools.partial(
    pl.kernel,
    mesh=_mesh,
    out_type=[
        jax.ShapeDtypeStruct((B, D), jnp.float32),
        jax.ShapeDtypeStruct((B, D), jnp.float32),
    ],
    scratch_types=[
        pltpu.VMEM((_BPW,), jnp.int32),
        pltpu.VMEM((_BPW, D), jnp.float32),
        pltpu.VMEM((_BPW,), jnp.int32),
        pltpu.VMEM((_BPW, D), jnp.float32),
        pltpu.SemaphoreType.DMA,
        pltpu.SemaphoreType.DMA,
    ],
    compiler_params=pltpu.CompilerParams(use_tc_tiling_on_sc=False),
)
def _sc_gather2(uidx_hbm, utab_hbm, iidx_hbm, itab_hbm, uout_hbm, iout_hbm,
                uidx_v, urows_v, iidx_v, irows_v, usem, isem):
    wid = lax.axis_index("s") * _NC + lax.axis_index("c")
    base = wid * _BPW
    pltpu.sync_copy(uidx_hbm.at[pl.ds(base, _BPW)], uidx_v)
    pltpu.sync_copy(iidx_hbm.at[pl.ds(base, _BPW)], iidx_v)
    cu = pltpu.async_copy(utab_hbm.at[uidx_v], urows_v, usem)
    ci = pltpu.async_copy(itab_hbm.at[iidx_v], irows_v, isem)
    cu.wait()
    pltpu.sync_copy(urows_v, uout_hbm.at[pl.ds(base, _BPW)])
    ci.wait()
    pltpu.sync_copy(irows_v, iout_hbm.at[pl.ds(base, _BPW)])


def _towers_body(ur_ref, ir_ref,
                 uW1_ref, ub1_ref, uW2_ref, ub2_ref, uW3_ref, ub3_ref,
                 iW1_ref, ib1_ref, iW2_ref, ib2_ref, iW3_ref, ib3_ref,
                 uo_ref, io_ref):
    def tower(x, W1, b1, W2, b2, W3, b3):
        h = jnp.maximum(jnp.dot(x, W1, preferred_element_type=jnp.float32) + b1, 0.0)
        h = jnp.maximum(jnp.dot(h, W2, preferred_element_type=jnp.float32) + b2, 0.0)
        return jnp.dot(h, W3, preferred_element_type=jnp.float32) + b3

    uo_ref[...] = tower(ur_ref[...], uW1_ref[...], ub1_ref[...], uW2_ref[...],
                        ub2_ref[...], uW3_ref[...], ub3_ref[...])
    io_ref[...] = tower(ir_ref[...], iW1_ref[...], ib1_ref[...], iW2_ref[...],
                        ib2_ref[...], iW3_ref[...], ib3_ref[...])


_BLK = 2048


def _towers(user_raw, item_raw, weights):
    nblk = B // _BLK
    row_spec = pl.BlockSpec((_BLK, D), lambda i: (i, 0))
    full = lambda shape: pl.BlockSpec(shape, lambda i: (0,) * len(shape))
    w_specs = [
        full((D, H1)), full((1, H1)), full((H1, H2)), full((1, H2)),
        full((H2, D)), full((1, D)),
        full((D, H1)), full((1, H1)), full((H1, H2)), full((1, H2)),
        full((H2, D)), full((1, D)),
    ]
    return pl.pallas_call(
        _towers_body,
        grid=(nblk,),
        in_specs=[row_spec, row_spec] + w_specs,
        out_specs=[row_spec, row_spec],
        out_shape=[
            jax.ShapeDtypeStruct((B, D), jnp.float32),
            jax.ShapeDtypeStruct((B, D), jnp.float32),
        ],
    )(user_raw, item_raw, *weights)


def kernel(user_id, movie_id, user_table, item_table,
           uW1, ub1, uW2, ub2, uW3, ub3,
           iW1, ib1, iW2, ib2, iW3, ib3):
    uid = user_id.astype(jnp.int32)
    mid = movie_id.astype(jnp.int32)
    user_raw, item_raw = _sc_gather2(uid, user_table, mid, item_table)
    weights = (uW1, ub1.reshape(1, H1), uW2, ub2.reshape(1, H2), uW3,
               ub3.reshape(1, D),
               iW1, ib1.reshape(1, H1), iW2, ib2.reshape(1, H2), iW3,
               ib3.reshape(1, D))
    user_emb, item_emb = _towers(user_raw, item_raw, weights)
    return (user_emb, item_emb)


# speedup vs baseline: 1.3922x; 1.3922x over previous
import functools

import jax
import jax.numpy as jnp
from jax import lax
from jax.experimental import pallas as pl
from jax.experimental.pallas import tpu as pltpu
from jax.experimental.pallas import tpu_sc as plsc

B = 16384
V1 = 1000001
VP = 1000008
D = 32
DP = 128
H1 = 128
H2 = 64

_NC = 2
_NS = 16
_NW = _NC * _NS
_BPW = B // _NW

_mesh = plsc.VectorSubcoreMesh(core_axis_name="c", subcore_axis_name="s")

_TCW = 4096
_TGRID = -(-VP // _TCW)


def _tpad_body(t_ref, o_ref):
    o_ref[:, :D] = t_ref[...].T
    o_ref[:, D:] = jnp.zeros((_TCW, DP - D), jnp.float32)


def _tpad(tt):
    return pl.pallas_call(
        _tpad_body,
        grid=(_TGRID,),
        in_specs=[pl.BlockSpec((D, _TCW), lambda c: (0, c))],
        out_specs=pl.BlockSpec((_TCW, DP), lambda c: (c, 0)),
        out_shape=jax.ShapeDtypeStruct((VP, DP), jnp.float32),
    )(tt)


@functools.partial(
    pl.kernel,
    mesh=_mesh,
    out_type=jax.ShapeDtypeStruct((B, DP), jnp.float32),
    scratch_types=[
        pltpu.VMEM((_BPW,), jnp.int32),
        pltpu.VMEM((_BPW, DP), jnp.float32),
        pltpu.SemaphoreType.DMA,
    ],
    compiler_params=pltpu.CompilerParams(use_tc_tiling_on_sc=True),
)
def _sc_gather(idx_hbm, tab_hbm, out_hbm, idx_v, rows_v, sem):
    wid = lax.axis_index("s") * _NC + lax.axis_index("c")
    base = wid * _BPW
    pltpu.sync_copy(idx_hbm.at[pl.ds(base, _BPW)], idx_v)
    pltpu.async_copy(tab_hbm.at[idx_v], rows_v, sem).wait()
    pltpu.sync_copy(rows_v, out_hbm.at[pl.ds(base, _BPW)])


def _towers_body(xu_ref, xi_ref,
                 uW1_ref, ub1_ref, uW2_ref, ub2_ref, uW3_ref, ub3_ref,
                 iW1_ref, ib1_ref, iW2_ref, ib2_ref, iW3_ref, ib3_ref,
                 uo_ref, io_ref):
    def tower(x, W1, b1, W2, b2, W3, b3):
        h = jnp.maximum(jnp.dot(x, W1, preferred_element_type=jnp.float32) + b1, 0.0)
        h = jnp.maximum(jnp.dot(h, W2, preferred_element_type=jnp.float32) + b2, 0.0)
        return jnp.dot(h, W3, preferred_element_type=jnp.float32) + b3

    uo_ref[...] = tower(xu_ref[...], uW1_ref[...], ub1_ref[...], uW2_ref[...],
                        ub2_ref[...], uW3_ref[...], ub3_ref[...])
    io_ref[...] = tower(xi_ref[...], iW1_ref[...], ib1_ref[...], iW2_ref[...],
                        ib2_ref[...], iW3_ref[...], ib3_ref[...])


_BLK = 2048


def _towers(xu, xi, weights):
    row_spec = pl.BlockSpec((_BLK, DP), lambda i: (i, 0))
    out_spec = pl.BlockSpec((_BLK, D), lambda i: (i, 0))
    full = lambda shape: pl.BlockSpec(shape, lambda i: (0,) * len(shape))
    w_specs = [
        full((DP, H1)), full((1, H1)), full((H1, H2)), full((1, H2)),
        full((H2, D)), full((1, D)),
        full((DP, H1)), full((1, H1)), full((H1, H2)), full((1, H2)),
        full((H2, D)), full((1, D)),
    ]
    return pl.pallas_call(
        _towers_body,
        grid=(B // _BLK,),
        in_specs=[row_spec, row_spec] + w_specs,
        out_specs=[out_spec, out_spec],
        out_shape=[
            jax.ShapeDtypeStruct((B, D), jnp.float32),
            jax.ShapeDtypeStruct((B, D), jnp.float32),
        ],
    )(xu, xi, *weights)


def kernel(user_id, movie_id, user_table, item_table,
           uW1, ub1, uW2, ub2, uW3, ub3,
           iW1, ib1, iW2, ib2, iW3, ib3):
    uid = user_id.astype(jnp.int32)
    mid = movie_id.astype(jnp.int32)
    utab = _tpad(user_table.T)
    itab = _tpad(item_table.T)
    xu = _sc_gather(uid, utab)
    xi = _sc_gather(mid, itab)
    weights = (jnp.pad(uW1, ((0, DP - D), (0, 0))), ub1.reshape(1, H1),
               uW2, ub2.reshape(1, H2), uW3, ub3.reshape(1, D),
               jnp.pad(iW1, ((0, DP - D), (0, 0))), ib1.reshape(1, H1),
               iW2, ib2.reshape(1, H2), iW3, ib3.reshape(1, D))
    return _towers(xu, xi, weights)
